# baseline (speedup 1.0000x reference)
import functools

import jax
import jax.numpy as jnp
from jax import lax
from jax.experimental import pallas as pl
from jax.experimental.pallas import tpu as pltpu
from jax.experimental.pallas import tpu_sc as plsc

N = 32768
NC = 1
NS = 16
L = 16
NW = NC * NS
CHUNK = N // NW


def _sc_branch_body(x_hbm, wb_hbm, out_hbm, xv, ov, wbv, sem):
    wid = lax.axis_index("s") * NC + lax.axis_index("c")
    base = wid * CHUNK
    cp_x = pltpu.make_async_copy(x_hbm.at[pl.ds(base, CHUNK)], xv, sem)
    cp_wb = pltpu.make_async_copy(wb_hbm, wbv, sem)
    cp_x.start()
    cp_wb.start()
    cp_x.wait()
    cp_wb.wait()
    w = wbv[pl.ds(0, L)]
    b = wbv[pl.ds(L, L)]
    for j in range(CHUNK // L):
        v = xv[pl.ds(j * L, L)]
        ov[pl.ds(j * L, L)] = jnp.maximum(v - 1.0, 0.0) * w + b
    pltpu.sync_copy(ov, out_hbm.at[pl.ds(base, CHUNK)])


@jax.jit
def _branch_module(x, wb):
    mesh = plsc.VectorSubcoreMesh(
        core_axis_name="c", subcore_axis_name="s", num_cores=NC
    )
    return pl.kernel(
        _sc_branch_body,
        mesh=mesh,
        out_type=jax.ShapeDtypeStruct((N,), jnp.float32),
        scratch_types=[
            pltpu.VMEM((CHUNK,), jnp.float32),
            pltpu.VMEM((CHUNK,), jnp.float32),
            pltpu.VMEM((2 * L,), jnp.float32),
            pltpu.SemaphoreType.DMA,
        ],
    )(x, wb)


def kernel(x, W1, b1, W2, b2):
    wb = jnp.concatenate(
        [jnp.broadcast_to(W1.reshape(1), (L,)), jnp.broadcast_to(b1, (L,))]
    )
    out = _branch_module(x.reshape(N), wb)
    return out.reshape(N, 1)


# speedup vs baseline: 1.0166x; 1.0166x over previous
import functools

import jax
import jax.numpy as jnp
from jax import lax
from jax.experimental import pallas as pl
from jax.experimental.pallas import tpu as pltpu
from jax.experimental.pallas import tpu_sc as plsc

N = 32768
NC = 1
NS = 16
L = 16
NW = NC * NS
CHUNK = N // NW


def _sc_branch_body(x_hbm, wb_hbm, out_hbm, xv, ov, wbv, sem):
    wid = lax.axis_index("s") * NC + lax.axis_index("c")
    base = wid * CHUNK
    cp_x = pltpu.make_async_copy(x_hbm.at[pl.ds(base, CHUNK)], xv, sem)
    cp_wb = pltpu.make_async_copy(wb_hbm, wbv, sem)
    cp_x.start()
    cp_wb.start()
    cp_x.wait()
    cp_wb.wait()
    w = wbv[pl.ds(0, L)]
    b = wbv[pl.ds(L, L)]

    UNROLL = 16

    def blk(j, carry):
        off = j * (UNROLL * L)
        for k in range(UNROLL):
            v = xv[pl.ds(off + k * L, L)]
            ov[pl.ds(off + k * L, L)] = jnp.maximum(v - 1.0, 0.0) * w + b
        return carry

    lax.fori_loop(0, CHUNK // (UNROLL * L), blk, 0)
    pltpu.sync_copy(ov, out_hbm.at[pl.ds(base, CHUNK)])


@jax.jit
def _branch_module(x, wb):
    mesh = plsc.VectorSubcoreMesh(
        core_axis_name="c", subcore_axis_name="s", num_cores=NC
    )
    return pl.kernel(
        _sc_branch_body,
        mesh=mesh,
        out_type=jax.ShapeDtypeStruct((N,), jnp.float32),
        scratch_types=[
            pltpu.VMEM((CHUNK,), jnp.float32),
            pltpu.VMEM((CHUNK,), jnp.float32),
            pltpu.VMEM((2 * L,), jnp.float32),
            pltpu.SemaphoreType.DMA,
        ],
    )(x, wb)


def kernel(x, W1, b1, W2, b2):
    wb = jnp.concatenate(
        [jnp.broadcast_to(W1.reshape(1), (L,)), jnp.broadcast_to(b1, (L,))]
    )
    out = _branch_module(x.reshape(N), wb)
    return out.reshape(N, 1)
